# Initial kernel scaffold; baseline (speedup 1.0000x reference)
import functools

import jax
import jax.numpy as jnp
from jax import lax
from jax.experimental import pallas as pl
from jax.experimental.pallas import tpu as pltpu
from jax.experimental.pallas import tpu_sc as plsc

D_MODEL = 64
NUM_CORES = 2
NUM_SUBCORES = 16
NW = NUM_CORES * NUM_SUBCORES
CHUNK = 800
N_CHUNKS = 8


def _gather_kernel(idx_hbm, table_hbm, out_hbm, idx_v, rows_v, gsem, osem):
    wid = lax.axis_index("s") * NUM_CORES + lax.axis_index("c")
    pltpu.sync_copy(idx_hbm.at[wid], idx_v)
    for c in range(N_CHUNKS):
        pltpu.async_copy(table_hbm.at[idx_v.at[c]], rows_v, gsem).wait()
        pltpu.async_copy(rows_v, out_hbm.at[wid, c], osem).wait()


@jax.jit
def _run(x_flat, table):
    mesh = plsc.VectorSubcoreMesh(core_axis_name="c", subcore_axis_name="s")
    k = functools.partial(
        pl.kernel,
        mesh=mesh,
        out_type=jax.ShapeDtypeStruct((NW, N_CHUNKS, CHUNK, D_MODEL), jnp.float32),
        scratch_types=[
            pltpu.VMEM((N_CHUNKS, CHUNK), jnp.int32),
            pltpu.VMEM((CHUNK, D_MODEL), jnp.float32),
            pltpu.SemaphoreType.DMA,
            pltpu.SemaphoreType.DMA,
        ],
    )(_gather_kernel)
    return k(x_flat.reshape(NW, N_CHUNKS, CHUNK), table)


def kernel(x, se):
    b, h = x.shape
    table = se.reshape(se.shape[0], D_MODEL)
    out = _run(x.reshape(-1), table)
    return out.reshape(b, h, 1, D_MODEL)


# speedup vs baseline: 4.8851x; 4.8851x over previous
import functools

import jax
import jax.numpy as jnp
from jax import lax
from jax.experimental import pallas as pl
from jax.experimental.pallas import tpu as pltpu
from jax.experimental.pallas import tpu_sc as plsc

D_MODEL = 64
NUM_CORES = 2
NUM_SUBCORES = 16
NW = NUM_CORES * NUM_SUBCORES
IVEC = 128
N_CHUNKS = 50


def _gather_kernel(idx_hbm, table_hbm, out_hbm, idx_v, rows_v, gsem, osem):
    wid = lax.axis_index("s") * NUM_CORES + lax.axis_index("c")
    pltpu.sync_copy(idx_hbm.at[wid], idx_v)

    def body(c, carry):
        pltpu.async_copy(table_hbm.at[idx_v.at[c]], rows_v, gsem).wait()
        pltpu.async_copy(rows_v, out_hbm.at[wid, c], osem).wait()
        return carry

    lax.fori_loop(0, N_CHUNKS, body, 0)


@jax.jit
def _run(x_flat, table):
    mesh = plsc.VectorSubcoreMesh(core_axis_name="c", subcore_axis_name="s")
    k = functools.partial(
        pl.kernel,
        mesh=mesh,
        compiler_params=pltpu.CompilerParams(use_tc_tiling_on_sc=False),
        out_type=jax.ShapeDtypeStruct((NW, N_CHUNKS, IVEC, D_MODEL), jnp.float32),
        scratch_types=[
            pltpu.VMEM((N_CHUNKS, IVEC), jnp.int32),
            pltpu.VMEM((IVEC, D_MODEL), jnp.float32),
            pltpu.SemaphoreType.DMA,
            pltpu.SemaphoreType.DMA,
        ],
    )(_gather_kernel)
    return k(x_flat.reshape(NW, N_CHUNKS, IVEC), table)


def kernel(x, se):
    b, h = x.shape
    table = se.reshape(se.shape[0], D_MODEL)
    out = _run(x.reshape(-1), table)
    return out.reshape(b, h, 1, D_MODEL)


# speedup vs baseline: 5.5769x; 1.1416x over previous
import functools

import jax
import jax.numpy as jnp
from jax import lax
from jax.experimental import pallas as pl
from jax.experimental.pallas import tpu as pltpu
from jax.experimental.pallas import tpu_sc as plsc

D_MODEL = 64
NUM_CORES = 2
NUM_SUBCORES = 16
NW = NUM_CORES * NUM_SUBCORES
IVEC = 128
G = 5
STEP_ROWS = G * IVEC
N_STEPS = 10


def _gather_kernel(idx_hbm, table_hbm, out_hbm, idx_v, rows_v, gsem, osem):
    wid = lax.axis_index("s") * NUM_CORES + lax.axis_index("c")
    pltpu.sync_copy(idx_hbm.at[wid], idx_v)

    def gather_descs(s, b):
        return [
            pltpu.make_async_copy(
                table_hbm.at[idx_v.at[s * G + j]],
                rows_v.at[b].at[pl.ds(j * IVEC, IVEC)],
                gsem,
            )
            for j in range(G)
        ]

    def store_desc(s, b):
        return pltpu.make_async_copy(rows_v.at[b], out_hbm.at[wid, s], osem)

    for d in gather_descs(0, 0):
        d.start()

    def body(s, carry):
        b = s % 2
        for d in gather_descs(s, b):
            d.wait()

        @pl.when(s >= 1)
        def _():
            store_desc(s - 1, 1 - b).wait()

        @pl.when(s + 1 < N_STEPS)
        def _():
            for d in gather_descs(s + 1, 1 - b):
                d.start()

        store_desc(s, b).start()
        return carry

    lax.fori_loop(0, N_STEPS, body, 0)
    store_desc(N_STEPS - 1, (N_STEPS - 1) % 2).wait()


@jax.jit
def _run(x_flat, table):
    mesh = plsc.VectorSubcoreMesh(core_axis_name="c", subcore_axis_name="s")
    k = functools.partial(
        pl.kernel,
        mesh=mesh,
        compiler_params=pltpu.CompilerParams(use_tc_tiling_on_sc=False),
        out_type=jax.ShapeDtypeStruct(
            (NW, N_STEPS, STEP_ROWS, D_MODEL), jnp.float32
        ),
        scratch_types=[
            pltpu.VMEM((N_STEPS * G, IVEC), jnp.int32),
            pltpu.VMEM((2, STEP_ROWS, D_MODEL), jnp.float32),
            pltpu.SemaphoreType.DMA,
            pltpu.SemaphoreType.DMA,
        ],
    )(_gather_kernel)
    return k(x_flat.reshape(NW, N_STEPS * G, IVEC), table)


def kernel(x, se):
    b, h = x.shape
    table = se.reshape(se.shape[0], D_MODEL)
    out = _run(x.reshape(-1), table)
    return out.reshape(b, h, 1, D_MODEL)
